# baseline (speedup 1.0000x reference)
import functools

import jax
import jax.numpy as jnp
from jax import lax
from jax.experimental import pallas as pl
from jax.experimental.pallas import tpu as pltpu
from jax.experimental.pallas import tpu_sc as plsc

BATCH = 1024
SEQ = 200
EMBED_DIM = 32
TOTAL = BATCH * SEQ

NC = 2
NS = 16
NW = NC * NS
CHUNK = 128

NQ = 4
BCH = BATCH // NQ
NSCH = NW // NQ
SCH = SEQ // NSCH

_mesh = plsc.VectorSubcoreMesh(core_axis_name="c", subcore_axis_name="s")

_LANE = lax.iota(jnp.int32, 16) if False else None


@functools.partial(
    pl.kernel,
    mesh=_mesh,
    compiler_params=pltpu.CompilerParams(
        use_tc_tiling_on_sc=False, needs_layout_passes=False
    ),
    out_type=jax.ShapeDtypeStruct((SEQ, EMBED_DIM, BATCH), jnp.float32),
    scratch_types=[
        pltpu.VMEM((SCH, BCH), jnp.int32),
        pltpu.VMEM((BCH, EMBED_DIM), jnp.float32),
        pltpu.VMEM((EMBED_DIM, BCH), jnp.float32),
        pltpu.SemaphoreType.DMA,
    ],
)
def _sc_gather(table_hbm, idx_hbm, out_hbm, idx_v, rows_v, outb_v, sem):
    wid = lax.axis_index("s") * NC + lax.axis_index("c")
    sch = wid // NQ
    q = wid % NQ
    s0 = sch * SCH
    b0 = q * BCH
    pltpu.sync_copy(idx_hbm.at[pl.ds(s0, SCH), pl.ds(b0, BCH)], idx_v)
    row_ids = [jnp.arange(g * 16, g * 16 + 16, dtype=jnp.int32)
               for g in range(BCH // 16)]
    col_ids = [jnp.full((16,), e, jnp.int32) for e in range(EMBED_DIM)]

    def per_seq(s_i, carry):
        for k in range(BCH // CHUNK):
            idx_chunk = idx_v.at[s_i, pl.ds(k * CHUNK, CHUNK)]
            pltpu.async_copy(
                table_hbm.at[idx_chunk],
                rows_v.at[pl.ds(k * CHUNK, CHUNK), :],
                sem,
            ).wait()

        for e in range(EMBED_DIM):
            for g in range(BCH // 16):
                vec = plsc.load_gather(rows_v, [row_ids[g], col_ids[e]])
                outb_v[e, g * 16:(g + 1) * 16] = vec

        pltpu.sync_copy(outb_v, out_hbm.at[s0 + s_i, :, pl.ds(b0, BCH)])
        return carry

    lax.fori_loop(0, SCH, per_seq, 0)


_TR_BLK = 128


def _tracker_body(tr_ref, ids_ref, out_ref):
    w = pl.program_id(0)
    t = tr_ref[...]
    out_ref[...] = t

    @pl.when(w < BATCH // _TR_BLK)
    def _():
        col = lax.broadcasted_iota(jnp.int32, (_TR_BLK, 256), 1)
        out_ref[:, :256] = jnp.where(col < SEQ, ids_ref[...], t[:, :256])


def _tracker(tr, ids_pad):
    n = tr.shape[0] // _TR_BLK
    return pl.pallas_call(
        _tracker_body,
        grid=(n,),
        in_specs=[
            pl.BlockSpec((_TR_BLK, tr.shape[1]), lambda w: (w, 0)),
            pl.BlockSpec((_TR_BLK, 256), lambda w: (jnp.minimum(w, BATCH // _TR_BLK - 1), 0)),
        ],
        out_specs=pl.BlockSpec((_TR_BLK, tr.shape[1]), lambda w: (w, 0)),
        out_shape=jax.ShapeDtypeStruct(tr.shape, jnp.int32),
    )(tr, ids_pad)


def kernel(inp_ids, table, idx_tracker):
    ids32 = inp_ids.astype(jnp.int32)
    ids_t = ids32.T
    out3 = _sc_gather(table, ids_t)
    out = jnp.transpose(out3, (2, 0, 1))
    ids_pad = jnp.pad(ids32, ((0, 0), (0, 256 - SEQ)))
    tracker = _tracker(idx_tracker.astype(jnp.int32), ids_pad).astype(idx_tracker.dtype)
    return out, tracker
